# Initial kernel scaffold; baseline (speedup 1.0000x reference)
import jax
import jax.numpy as jnp
from jax.experimental import pallas as pl


def kernel(coords, atom_types, residue_types, meta_classes, W_coord, b_coord, atom_table, residue_table, meta_table):
    raise NotImplementedError("write your pallas kernel here")


# speedup vs baseline: 14.3298x; 14.3298x over previous
import functools
import jax
import jax.numpy as jnp
from jax import lax
from jax.experimental import pallas as pl
from jax.experimental.pallas import tpu as pltpu

_TB = 2048


def _body(coords_ref, at_ref, rt_ref, mt_ref, W_ref, b_ref,
          atab_ref, rtab_ref, mtab_ref, out_ref):
    proj = lax.dot_general(coords_ref[...], W_ref[...],
                           (((0,), (0,)), ((), ())),
                           preferred_element_type=jnp.float32)
    proj = proj + b_ref[...]
    h = jax.nn.silu(proj)

    def onehot_dot(ids, tab, v):
        oh = (ids[:, None] == lax.broadcasted_iota(jnp.int32, (_TB, v), 1)
              ).astype(jnp.float32)
        return jnp.dot(oh, tab, preferred_element_type=jnp.float32)

    h = h + onehot_dot(at_ref[0, 0, :], atab_ref[...], 128)
    h = h + onehot_dot(rt_ref[0, 0, :], rtab_ref[...], 32)
    h = h + onehot_dot(mt_ref[0, 0, :], mtab_ref[...], 16)
    out_ref[...] = h


def kernel(coords, atom_types, residue_types, meta_classes, W_coord, b_coord,
           atom_table, residue_table, meta_table):
    B, L, D = coords.shape[0], coords.shape[1], W_coord.shape[1]
    N = B * L
    G = N // _TB
    coords_t = coords.reshape(N, 3).T
    at = atom_types.reshape(G, 1, _TB)
    rt = residue_types.reshape(G, 1, _TB)
    mt = meta_classes.reshape(G, 1, _TB)
    b2 = b_coord.reshape(1, D)

    out = pl.pallas_call(
        _body,
        grid=(G,),
        in_specs=[
            pl.BlockSpec((3, _TB), lambda i: (0, i)),
            pl.BlockSpec((1, 1, _TB), lambda i: (i, 0, 0)),
            pl.BlockSpec((1, 1, _TB), lambda i: (i, 0, 0)),
            pl.BlockSpec((1, 1, _TB), lambda i: (i, 0, 0)),
            pl.BlockSpec((3, D), lambda i: (0, 0)),
            pl.BlockSpec((1, D), lambda i: (0, 0)),
            pl.BlockSpec((128, D), lambda i: (0, 0)),
            pl.BlockSpec((32, D), lambda i: (0, 0)),
            pl.BlockSpec((16, D), lambda i: (0, 0)),
        ],
        out_specs=pl.BlockSpec((_TB, D), lambda i: (i, 0)),
        out_shape=jax.ShapeDtypeStruct((N, D), jnp.float32),
    )(coords_t, at, rt, mt, W_coord, b2,
      atom_table, residue_table, meta_table)
    return out.reshape(B, L, D)
